# baseline (device time: 129145 ns/iter reference)
import jax
import jax.numpy as jnp
from jax import lax
from jax.experimental import pallas as pl
from jax.experimental.pallas import tpu as pltpu

KQ = 8


def kernel(partial, resid, gamma):
    m, d = resid.shape
    quarter = m // 4
    mb = quarter // KQ
    x2 = partial.reshape(m, d)
    gamma2 = gamma.reshape(1, d)

    def body(p_ref, r_hbm, g_ref, out_ref, r_buf,
             copy_sems, z_send, z_recv, x_send, x_recv, y_send, y_recv):
        my_x = lax.axis_index("x")
        my_y = lax.axis_index("y")
        my_z = lax.axis_index("z")
        qb = (2 * my_x + my_y) * quarter
        xqb = (2 * (1 - my_x) + my_y) * quarter
        yqb = (2 * my_x + (1 - my_y)) * quarter
        dqb = (2 * (1 - my_x) + (1 - my_y)) * quarter
        zpeer = (my_x, my_y, 1 - my_z)
        xnbr = (1 - my_x, my_y, my_z)
        ynbr = (my_x, 1 - my_y, my_z)

        barrier_sem = pltpu.get_barrier_semaphore()
        for nbr in (zpeer, xnbr, ynbr):
            pl.semaphore_signal(
                barrier_sem, inc=1, device_id=nbr,
                device_id_type=pl.DeviceIdType.MESH)
        pl.semaphore_wait(barrier_sem, 3)

        def remote(src_rows, dst_rows, send_sem, recv_sem, dev):
            return pltpu.make_async_remote_copy(
                src_ref=out_ref.at[src_rows, :],
                dst_ref=out_ref.at[dst_rows, :],
                send_sem=send_sem, recv_sem=recv_sem,
                device_id=dev, device_id_type=pl.DeviceIdType.MESH)

        z_rdmas = []
        for j in range(KQ):
            rows = pl.ds(qb + j * mb, mb)
            r = pltpu.make_async_remote_copy(
                src_ref=p_ref.at[rows, :], dst_ref=out_ref.at[rows, :],
                send_sem=z_send.at[j], recv_sem=z_recv.at[j],
                device_id=zpeer, device_id_type=pl.DeviceIdType.MESH)
            r.start()
            z_rdmas.append(r)

        r_copy = pltpu.make_async_copy(
            r_hbm.at[pl.ds(qb, quarter), :], r_buf, copy_sems.at[0])
        r_copy.start()
        r_copy.wait()

        x_sends, y_sends = [], []
        x_dsts, y_dsts = [], []
        for j in range(KQ):
            rows = pl.ds(qb + j * mb, mb)
            xrows = pl.ds(xqb + j * mb, mb)
            yrows = pl.ds(yqb + j * mb, mb)

            z_rdmas[j].wait_recv()
            y = p_ref[rows, :] + out_ref[rows, :] + r_buf[pl.ds(j * mb, mb), :]
            rms = jnp.sqrt(jnp.mean(y * y, axis=-1, keepdims=True) + 1e-6)
            out_ref[rows, :] = y / rms * g_ref[...]

            for sems, dev, lst in ((x_send, xnbr, x_sends),
                                   (y_send, ynbr, y_sends)):
                s = remote(rows, rows, sems.at[j],
                           (x_recv if dev is xnbr else y_recv).at[j], dev)
                s.start()
                lst.append(s)

            xr = remote(xrows, xrows, x_send.at[j], x_recv.at[j], xnbr)
            xr.wait_recv()
            x_dsts.append(xr)
            if j % 2 == 1:
                f = remote(xrows, xrows, y_send.at[KQ + j // 2],
                           y_recv.at[KQ + j // 2], ynbr)
                f.start()
                y_sends.append(f)
            yr = remote(yrows, yrows, y_send.at[j], y_recv.at[j], ynbr)
            yr.wait_recv()
            y_dsts.append(yr)
            if j % 2 == 0:
                f = remote(yrows, yrows, x_send.at[KQ + j // 2],
                           x_recv.at[KQ + j // 2], xnbr)
                f.start()
                x_sends.append(f)

        for j in range(KQ):
            drows = pl.ds(dqb + j * mb, mb)
            sems = x_recv if j % 2 == 0 else y_recv
            dr = remote(drows, drows, (x_send if j % 2 == 0 else y_send).at[KQ + j // 2],
                        sems.at[KQ + j // 2],
                        xnbr if j % 2 == 0 else ynbr)
            dr.wait_recv()
        for r in z_rdmas + x_sends + y_sends:
            r.wait_send()

    return pl.pallas_call(
        body,
        out_shape=jax.ShapeDtypeStruct((m, d), jnp.float32),
        in_specs=[
            pl.BlockSpec(memory_space=pltpu.VMEM),
            pl.BlockSpec(memory_space=pl.ANY),
            pl.BlockSpec(memory_space=pltpu.VMEM),
        ],
        out_specs=pl.BlockSpec(memory_space=pltpu.VMEM),
        scratch_shapes=[
            pltpu.VMEM((quarter, d), jnp.float32),
            pltpu.SemaphoreType.DMA((2,)),
            pltpu.SemaphoreType.DMA((KQ,)),
            pltpu.SemaphoreType.DMA((KQ,)),
            pltpu.SemaphoreType.DMA((KQ + KQ // 2,)),
            pltpu.SemaphoreType.DMA((KQ + KQ // 2,)),
            pltpu.SemaphoreType.DMA((KQ + KQ // 2,)),
            pltpu.SemaphoreType.DMA((KQ + KQ // 2,)),
        ],
        compiler_params=pltpu.CompilerParams(collective_id=0),
    )(x2, resid, gamma2)


# device time: 93391 ns/iter; 1.3828x vs baseline; 1.3828x over previous
import jax
import jax.numpy as jnp
from jax import lax
from jax.experimental import pallas as pl
from jax.experimental.pallas import tpu as pltpu

KQ = 8


def kernel(partial, resid, gamma):
    m, d = resid.shape
    quarter = m // 4
    mb = quarter // KQ
    x2 = partial.reshape(m, d)
    gamma2 = gamma.reshape(1, d)

    def body(p_ref, r_hbm, g_ref, out_ref, r_buf,
             copy_sems, z_send, z_recv, x_send, x_recv, y_send, y_recv):
        my_x = lax.axis_index("x")
        my_y = lax.axis_index("y")
        my_z = lax.axis_index("z")
        qb = (2 * my_x + my_y) * quarter
        xqb = (2 * (1 - my_x) + my_y) * quarter
        yqb = (2 * my_x + (1 - my_y)) * quarter
        dqb = (2 * (1 - my_x) + (1 - my_y)) * quarter
        zpeer = (my_x, my_y, 1 - my_z)
        xnbr = (1 - my_x, my_y, my_z)
        ynbr = (my_x, 1 - my_y, my_z)

        barrier_sem = pltpu.get_barrier_semaphore()
        for nbr in (zpeer, xnbr, ynbr):
            pl.semaphore_signal(
                barrier_sem, inc=1, device_id=nbr,
                device_id_type=pl.DeviceIdType.MESH)
        pl.semaphore_wait(barrier_sem, 3)

        def remote(src_rows, dst_rows, send_sem, recv_sem, dev):
            return pltpu.make_async_remote_copy(
                src_ref=out_ref.at[src_rows, :],
                dst_ref=out_ref.at[dst_rows, :],
                send_sem=send_sem, recv_sem=recv_sem,
                device_id=dev, device_id_type=pl.DeviceIdType.MESH)

        z_rdmas = []
        for j in range(KQ):
            rows = pl.ds(qb + j * mb, mb)
            r = pltpu.make_async_remote_copy(
                src_ref=p_ref.at[rows, :], dst_ref=out_ref.at[rows, :],
                send_sem=z_send.at[j], recv_sem=z_recv.at[j],
                device_id=zpeer, device_id_type=pl.DeviceIdType.MESH)
            r.start()
            z_rdmas.append(r)

        r_copy = pltpu.make_async_copy(
            r_hbm.at[pl.ds(qb, quarter), :], r_buf, copy_sems.at[0])
        r_copy.start()
        r_copy.wait()

        x_sends, y_sends = [], []
        for j in range(KQ):
            rows = pl.ds(qb + j * mb, mb)
            z_rdmas[j].wait_recv()
            y = p_ref[rows, :] + out_ref[rows, :] + r_buf[pl.ds(j * mb, mb), :]
            rms = jnp.sqrt(jnp.mean(y * y, axis=-1, keepdims=True) + 1e-6)
            out_ref[rows, :] = y / rms * g_ref[...]
            for sems, rsems, dev, lst in ((x_send, x_recv, xnbr, x_sends),
                                          (y_send, y_recv, ynbr, y_sends)):
                s = remote(rows, rows, sems.at[j], rsems.at[j], dev)
                s.start()
                lst.append(s)

        for j in range(KQ):
            xrows = pl.ds(xqb + j * mb, mb)
            yrows = pl.ds(yqb + j * mb, mb)
            remote(xrows, xrows, x_send.at[j], x_recv.at[j], xnbr).wait_recv()
            if j % 2 == 1:
                f = remote(xrows, xrows, y_send.at[KQ + j // 2],
                           y_recv.at[KQ + j // 2], ynbr)
                f.start()
                y_sends.append(f)
            remote(yrows, yrows, y_send.at[j], y_recv.at[j], ynbr).wait_recv()
            if j % 2 == 0:
                f = remote(yrows, yrows, x_send.at[KQ + j // 2],
                           x_recv.at[KQ + j // 2], xnbr)
                f.start()
                x_sends.append(f)

        for j in range(KQ):
            drows = pl.ds(dqb + j * mb, mb)
            sems = x_recv if j % 2 == 0 else y_recv
            dr = remote(drows, drows, (x_send if j % 2 == 0 else y_send).at[KQ + j // 2],
                        sems.at[KQ + j // 2],
                        xnbr if j % 2 == 0 else ynbr)
            dr.wait_recv()
        for r in z_rdmas + x_sends + y_sends:
            r.wait_send()

    return pl.pallas_call(
        body,
        out_shape=jax.ShapeDtypeStruct((m, d), jnp.float32),
        in_specs=[
            pl.BlockSpec(memory_space=pltpu.VMEM),
            pl.BlockSpec(memory_space=pl.ANY),
            pl.BlockSpec(memory_space=pltpu.VMEM),
        ],
        out_specs=pl.BlockSpec(memory_space=pltpu.VMEM),
        scratch_shapes=[
            pltpu.VMEM((quarter, d), jnp.float32),
            pltpu.SemaphoreType.DMA((2,)),
            pltpu.SemaphoreType.DMA((KQ,)),
            pltpu.SemaphoreType.DMA((KQ,)),
            pltpu.SemaphoreType.DMA((KQ + KQ // 2,)),
            pltpu.SemaphoreType.DMA((KQ + KQ // 2,)),
            pltpu.SemaphoreType.DMA((KQ + KQ // 2,)),
            pltpu.SemaphoreType.DMA((KQ + KQ // 2,)),
        ],
        compiler_params=pltpu.CompilerParams(collective_id=0),
    )(x2, resid, gamma2)
